# baseline (speedup 1.0000x reference)
import dataclasses
import functools

import jax
import jax.numpy as jnp
from jax import lax
from jax.experimental import pallas as pl
from jax.experimental.pallas import tpu as pltpu
from jax.experimental.pallas import tpu_sc as plsc

N = 10000
D = 256
DH = 128
NROW = 10240
E = 160000
CHUNK = 128
NCHUNK = 80
GB = 8
NG = NCHUNK // GB
EPAD = 16 * NCHUNK * CHUNK
TRASH = 10200
BN = 1024
NSUB = 16
RPS = NROW // NSUB

_MESH = dict(core_axis_name="c", subcore_axis_name="s")


def _sc_compiler_params():
    cp = pltpu.CompilerParams()
    if "needs_layout_passes" in pltpu.CompilerParams.__dataclass_fields__:
        cp = dataclasses.replace(cp, needs_layout_passes=False)
    return cp


def _sc_degree(dst3):
    mesh = plsc.VectorSubcoreMesh(**_MESH)

    @functools.partial(
        pl.kernel,
        mesh=mesh,
        out_type=jax.ShapeDtypeStruct((2 * NSUB, RPS), jnp.float32),
        compiler_params=_sc_compiler_params(),
        scratch_types=[
            pltpu.VMEM((NCHUNK, CHUNK), jnp.int32),
            pltpu.VMEM((NROW,), jnp.float32),
            pltpu.VMEM((NSUB, RPS), jnp.float32),
            pltpu.VMEM((RPS,), jnp.float32),
            pltpu.VMEM_SHARED((NSUB, NROW), jnp.float32),
        ],
    )
    def k(dst_hbm, out_hbm, dst_v, hist_v, red_v, comp_v, acc):
        cid = lax.axis_index("c")
        sid = lax.axis_index("s")
        one16 = jnp.ones((16,), jnp.float32)
        zero16 = jnp.zeros((16,), jnp.float32)

        @pl.loop(0, NROW // 16)
        def _(i):
            hist_v[pl.ds(i * 16, 16)] = zero16

        pltpu.sync_copy(dst_hbm.at[sid], dst_v)

        half = NCHUNK // 2

        @pl.loop(cid * half, (cid + 1) * half)
        def _(j):
            for kk in range(CHUNK // 16):
                idx = dst_v[j, pl.ds(kk * 16, 16)]
                plsc.addupdate_scatter(hist_v, [idx], one16)

        pltpu.sync_copy(hist_v, acc.at[sid])
        plsc.subcore_barrier()

        for k16 in range(NSUB):
            pltpu.sync_copy(acc.at[k16].at[pl.ds(sid * RPS, RPS)],
                            red_v.at[k16])
        for cc in range(RPS // 16):
            tot = red_v[0, pl.ds(cc * 16, 16)]
            for k16 in range(1, NSUB):
                tot = tot + red_v[k16, pl.ds(cc * 16, 16)]
            comp_v[pl.ds(cc * 16, 16)] = tot

        pltpu.sync_copy(comp_v, out_hbm.at[cid * NSUB + sid])

    return k(dst3)


def _sc_scatter(g, sd, zeros):
    mesh = plsc.VectorSubcoreMesh(**_MESH)

    @functools.partial(
        pl.kernel,
        mesh=mesh,
        out_type=jax.ShapeDtypeStruct((2, NROW, DH), jnp.float32),
        scratch_types=[
            pltpu.VMEM((2 * GB, CHUNK), jnp.int32),
            pltpu.VMEM((2 * GB, CHUNK), jnp.int32),
            pltpu.VMEM((CHUNK, DH), jnp.float32),
            pltpu.VMEM((CHUNK, DH), jnp.float32),
            pltpu.VMEM_SHARED((NROW, DH), jnp.float32),
            pltpu.SemaphoreType.DMA,
            pltpu.SemaphoreType.DMA,
            pltpu.SemaphoreType.DMA,
            pltpu.SemaphoreType.DMA,
            pltpu.SemaphoreType.DMA,
            pltpu.SemaphoreType.DMA,
        ],
    )
    def k(g_hbm, sd_hbm, z_hbm, out_hbm,
          sd_a, sd_b, rows_a, rows_b, acc, sia, sib, sga, sgb, ssa, ssb):
        cid = lax.axis_index("c")
        sid = lax.axis_index("s")
        pltpu.sync_copy(z_hbm.at[pl.ds(sid * RPS, RPS)],
                        acc.at[pl.ds(sid * RPS, RPS)])
        plsc.subcore_barrier()

        gtab = g_hbm.at[cid]

        def idx_cp(grp, buf, sem):
            return pltpu.make_async_copy(sd_hbm.at[sid * NG + grp], buf, sem)

        def gcp(sdbuf, jj, rbuf, sem):
            return pltpu.make_async_copy(gtab.at[sdbuf.at[jj]], rbuf, sem)

        def scp(sdbuf, jj, rbuf, sem):
            return pltpu.make_async_copy(rbuf, acc.at[sdbuf.at[GB + jj]], sem)

        idx_cp(0, sd_a, sia).start()

        def group(grp, sdc, sic, sdn, sin):
            idx_cp(grp, sdc, sic).wait()

            @pl.when(grp + 1 < NG)
            def _():
                idx_cp(grp + 1, sdn, sin).start()

            gcp(sdc, 0, rows_a, sga).start()
            for jj in range(GB):
                even = jj % 2 == 0
                cur, gcur, scur = ((rows_a, sga, ssa) if even
                                   else (rows_b, sgb, ssb))
                oth, goth, soth = ((rows_b, sgb, ssb) if even
                                   else (rows_a, sga, ssa))
                gcp(sdc, jj, cur, gcur).wait()
                if jj + 1 < GB:
                    if jj >= 1:
                        scp(sdc, jj - 1, oth, soth).wait()
                    gcp(sdc, jj + 1, oth, goth).start()
                scp(sdc, jj, cur, scur).start(add=True)
            scp(sdc, GB - 2, rows_a, ssa).wait()
            scp(sdc, GB - 1, rows_b, ssb).wait()

        @pl.loop(0, NG, step=2)
        def _(grp):
            group(grp, sd_a, sia, sd_b, sib)
            group(grp + 1, sd_b, sib, sd_a, sia)

        plsc.subcore_barrier()
        pltpu.sync_copy(acc.at[pl.ds(sid * RPS, RPS)],
                        out_hbm.at[cid].at[pl.ds(sid * RPS, RPS)])

    return k(g, sd, zeros)


def _tc_matmul_split(x, w_split):
    def body(x_ref, w_ref, o_ref):
        o_ref[0] = jnp.dot(x_ref[...], w_ref[0],
                           preferred_element_type=jnp.float32,
                           precision=lax.Precision.HIGHEST)

    return pl.pallas_call(
        body,
        grid=(NROW // BN, 2),
        in_specs=[pl.BlockSpec((BN, D), lambda i, c: (i, 0)),
                  pl.BlockSpec((1, D, DH), lambda i, c: (c, 0, 0))],
        out_specs=pl.BlockSpec((1, BN, DH), lambda i, c: (c, i, 0)),
        out_shape=jax.ShapeDtypeStruct((2, NROW, DH), jnp.float32),
    )(x, w_split)


def _tc_scale(deg2, h):
    def body(dg_ref, h_ref, di_ref, g_ref):
        d = dg_ref[0] + dg_ref[1] + 1.0
        di = lax.rsqrt(d)
        di_ref[...] = di
        g_ref[0] = h_ref[0] * di
        g_ref[1] = h_ref[1] * di

    return pl.pallas_call(
        body,
        grid=(NROW // BN,),
        in_specs=[pl.BlockSpec((2, BN, 1), lambda i: (0, i, 0)),
                  pl.BlockSpec((2, BN, DH), lambda i: (0, i, 0))],
        out_specs=[pl.BlockSpec((BN, 1), lambda i: (i, 0)),
                   pl.BlockSpec((2, BN, DH), lambda i: (0, i, 0))],
        out_shape=[jax.ShapeDtypeStruct((NROW, 1), jnp.float32),
                   jax.ShapeDtypeStruct((2, NROW, DH), jnp.float32)],
    )(deg2, h)


def _tc_layer2(s1, g1, di, b1_split, w2_split):
    def body(s_ref, g_ref, di_ref, b_ref, w_ref, o_ref):
        d = di_ref[...]
        t0 = jax.nn.relu((s_ref[0] + g_ref[0]) * d + b_ref[0])
        t1 = jax.nn.relu((s_ref[1] + g_ref[1]) * d + b_ref[1])
        t = jnp.concatenate([t0, t1], axis=1)
        o_ref[0] = jnp.dot(t, w_ref[0],
                           preferred_element_type=jnp.float32,
                           precision=lax.Precision.HIGHEST) * d

    return pl.pallas_call(
        body,
        grid=(NROW // BN, 2),
        in_specs=[pl.BlockSpec((2, BN, DH), lambda i, c: (0, i, 0)),
                  pl.BlockSpec((2, BN, DH), lambda i, c: (0, i, 0)),
                  pl.BlockSpec((BN, 1), lambda i, c: (i, 0)),
                  pl.BlockSpec((2, 1, DH), lambda i, c: (0, 0, 0)),
                  pl.BlockSpec((1, D, DH), lambda i, c: (c, 0, 0))],
        out_specs=pl.BlockSpec((1, BN, DH), lambda i, c: (c, i, 0)),
        out_shape=jax.ShapeDtypeStruct((2, NROW, DH), jnp.float32),
    )(s1, g1, di, b1_split, w2_split)


def _tc_final(s2, g2, di, b2_split):
    def body(s_ref, g_ref, di_ref, b_ref, o_ref):
        d = di_ref[...]
        o0 = (s_ref[0] + g_ref[0]) * d + b_ref[0]
        o1 = (s_ref[1] + g_ref[1]) * d + b_ref[1]
        o_ref[...] = jnp.concatenate([o0, o1], axis=1)

    return pl.pallas_call(
        body,
        grid=(NROW // BN,),
        in_specs=[pl.BlockSpec((2, BN, DH), lambda i: (0, i, 0)),
                  pl.BlockSpec((2, BN, DH), lambda i: (0, i, 0)),
                  pl.BlockSpec((BN, 1), lambda i: (i, 0)),
                  pl.BlockSpec((2, 1, DH), lambda i: (0, 0, 0))],
        out_specs=pl.BlockSpec((BN, D), lambda i: (i, 0)),
        out_shape=jax.ShapeDtypeStruct((NROW, D), jnp.float32),
    )(s2, g2, di, b2_split)


def kernel(x, adj_t, W1, b1, W2, b2):
    src = adj_t[0].astype(jnp.int32)
    dst = adj_t[1].astype(jnp.int32)
    pad = EPAD - E
    src_p = jnp.concatenate([src, jnp.zeros((pad,), jnp.int32)])
    dst_p = jnp.concatenate([dst, jnp.full((pad,), TRASH, jnp.int32)])
    dst3 = dst_p.reshape(NSUB, NCHUNK, CHUNK)
    sd = jnp.concatenate([src_p.reshape(NSUB, NG, GB, CHUNK),
                          dst_p.reshape(NSUB, NG, GB, CHUNK)],
                         axis=2).reshape(NSUB * NG, 2 * GB, CHUNK)

    x_p = jnp.pad(x, ((0, NROW - N), (0, 0)))
    w1s = W1.reshape(D, 2, DH).transpose(1, 0, 2)
    w2s = W2.reshape(D, 2, DH).transpose(1, 0, 2)
    b1s = b1.reshape(2, 1, DH)
    b2s = b2.reshape(2, 1, DH)
    zeros_big = jnp.zeros((NROW, DH), jnp.float32)

    deg_parts = _sc_degree(dst3)
    deg2 = deg_parts.reshape(2, NROW, 1)
    h1 = _tc_matmul_split(x_p, w1s)
    di, g1 = _tc_scale(deg2, h1)
    s1 = _sc_scatter(g1, sd, zeros_big)
    g2 = _tc_layer2(s1, g1, di, b1s, w2s)
    s2 = _sc_scatter(g2, sd, zeros_big)
    out = _tc_final(s2, g2, di, b2s)
    return out[:N]


# speedup vs baseline: 1.0309x; 1.0309x over previous
import dataclasses
import functools

import jax
import jax.numpy as jnp
from jax import lax
from jax.experimental import pallas as pl
from jax.experimental.pallas import tpu as pltpu
from jax.experimental.pallas import tpu_sc as plsc

N = 10000
D = 256
DH = 128
NROW = 10240
E = 160000
CHUNK = 128
NCHUNK = 80
GB = 8
NG = NCHUNK // GB
EPAD = 16 * NCHUNK * CHUNK
TRASH = 10200
BN = 1024
NSUB = 16
RPS = NROW // NSUB

_MESH = dict(core_axis_name="c", subcore_axis_name="s")


def _sc_compiler_params():
    cp = pltpu.CompilerParams()
    if "needs_layout_passes" in pltpu.CompilerParams.__dataclass_fields__:
        cp = dataclasses.replace(cp, needs_layout_passes=False)
    return cp


def _sc_degree(dst3):
    mesh = plsc.VectorSubcoreMesh(**_MESH)

    @functools.partial(
        pl.kernel,
        mesh=mesh,
        out_type=jax.ShapeDtypeStruct((2 * NSUB, RPS), jnp.float32),
        compiler_params=_sc_compiler_params(),
        scratch_types=[
            pltpu.VMEM((NCHUNK, CHUNK), jnp.int32),
            pltpu.VMEM((NROW,), jnp.float32),
            pltpu.VMEM((NSUB, RPS), jnp.float32),
            pltpu.VMEM((RPS,), jnp.float32),
            pltpu.VMEM_SHARED((NSUB, NROW), jnp.float32),
        ],
    )
    def k(dst_hbm, out_hbm, dst_v, hist_v, red_v, comp_v, acc):
        cid = lax.axis_index("c")
        sid = lax.axis_index("s")
        one16 = jnp.ones((16,), jnp.float32)
        zero16 = jnp.zeros((16,), jnp.float32)

        @pl.loop(0, NROW // 16)
        def _(i):
            hist_v[pl.ds(i * 16, 16)] = zero16

        pltpu.sync_copy(dst_hbm.at[sid], dst_v)

        half = NCHUNK // 2

        @pl.loop(cid * half, (cid + 1) * half)
        def _(j):
            for kk in range(CHUNK // 16):
                idx = dst_v[j, pl.ds(kk * 16, 16)]
                plsc.addupdate_scatter(hist_v, [idx], one16)

        pltpu.sync_copy(hist_v, acc.at[sid])
        plsc.subcore_barrier()

        for k16 in range(NSUB):
            pltpu.sync_copy(acc.at[k16].at[pl.ds(sid * RPS, RPS)],
                            red_v.at[k16])
        for cc in range(RPS // 16):
            tot = red_v[0, pl.ds(cc * 16, 16)]
            for k16 in range(1, NSUB):
                tot = tot + red_v[k16, pl.ds(cc * 16, 16)]
            comp_v[pl.ds(cc * 16, 16)] = tot

        pltpu.sync_copy(comp_v, out_hbm.at[cid * NSUB + sid])

    return k(dst3)


def _sc_scatter(g, sd, zeros):
    mesh = plsc.VectorSubcoreMesh(**_MESH)

    @functools.partial(
        pl.kernel,
        mesh=mesh,
        out_type=jax.ShapeDtypeStruct((2, NROW, DH), jnp.float32),
        scratch_types=[
            pltpu.VMEM((2 * GB, CHUNK), jnp.int32),
            pltpu.VMEM((2 * GB, CHUNK), jnp.int32),
            pltpu.VMEM((CHUNK, DH), jnp.float32),
            pltpu.VMEM((CHUNK, DH), jnp.float32),
            pltpu.VMEM_SHARED((NROW, DH), jnp.float32),
            pltpu.SemaphoreType.DMA,
            pltpu.SemaphoreType.DMA,
            pltpu.SemaphoreType.DMA,
            pltpu.SemaphoreType.DMA,
            pltpu.SemaphoreType.DMA,
            pltpu.SemaphoreType.DMA,
        ],
    )
    def k(g_hbm, sd_hbm, z_hbm, out_hbm,
          sd_a, sd_b, rows_a, rows_b, acc, sia, sib, sga, sgb, ssa, ssb):
        cid = lax.axis_index("c")
        sid = lax.axis_index("s")
        pltpu.sync_copy(z_hbm.at[pl.ds(sid * RPS, RPS)],
                        acc.at[pl.ds(sid * RPS, RPS)])
        plsc.subcore_barrier()

        gtab = g_hbm.at[cid]

        def idx_cp(grp, buf, sem):
            return pltpu.make_async_copy(sd_hbm.at[sid * NG + grp], buf, sem)

        def gcp(sdbuf, jj, rbuf, sem):
            return pltpu.make_async_copy(gtab.at[sdbuf.at[jj]], rbuf, sem)

        def scp(sdbuf, jj, rbuf, sem):
            return pltpu.make_async_copy(rbuf, acc.at[sdbuf.at[GB + jj]], sem)

        idx_cp(0, sd_a, sia).start()

        def group(grp, sdc, sic, sdn, sin):
            idx_cp(grp, sdc, sic).wait()

            @pl.when(grp + 1 < NG)
            def _():
                idx_cp(grp + 1, sdn, sin).start()

            gcp(sdc, 0, rows_a, sga).start()
            for jj in range(GB):
                even = jj % 2 == 0
                cur, gcur, scur = ((rows_a, sga, ssa) if even
                                   else (rows_b, sgb, ssb))
                oth, goth, soth = ((rows_b, sgb, ssb) if even
                                   else (rows_a, sga, ssa))
                gcp(sdc, jj, cur, gcur).wait()
                if jj + 1 < GB:
                    if jj >= 1:
                        scp(sdc, jj - 1, oth, soth).wait()
                    gcp(sdc, jj + 1, oth, goth).start()
                scp(sdc, jj, cur, scur).start(add=True)
            scp(sdc, GB - 2, rows_a, ssa).wait()
            scp(sdc, GB - 1, rows_b, ssb).wait()

        @pl.loop(0, NG, step=2)
        def _(grp):
            group(grp, sd_a, sia, sd_b, sib)
            group(grp + 1, sd_b, sib, sd_a, sia)

        plsc.subcore_barrier()
        pltpu.sync_copy(acc.at[pl.ds(sid * RPS, RPS)],
                        out_hbm.at[cid].at[pl.ds(sid * RPS, RPS)])

    return k(g, sd, zeros)


def _tc_mm_scale(x, w_split, deg2):
    def body(x_ref, w_ref, dg_ref, di_ref, g_ref):
        d = dg_ref[0] + dg_ref[1] + 1.0
        di = lax.rsqrt(d)
        di_ref[...] = di
        xb = x_ref[...]
        g_ref[0] = jnp.dot(xb, w_ref[0],
                           preferred_element_type=jnp.float32) * di
        g_ref[1] = jnp.dot(xb, w_ref[1],
                           preferred_element_type=jnp.float32) * di

    return pl.pallas_call(
        body,
        grid=(NROW // BN,),
        in_specs=[pl.BlockSpec((BN, D), lambda i: (i, 0)),
                  pl.BlockSpec((2, D, DH), lambda i: (0, 0, 0)),
                  pl.BlockSpec((2, BN, 1), lambda i: (0, i, 0))],
        out_specs=[pl.BlockSpec((BN, 1), lambda i: (i, 0)),
                   pl.BlockSpec((2, BN, DH), lambda i: (0, i, 0))],
        out_shape=[jax.ShapeDtypeStruct((NROW, 1), jnp.float32),
                   jax.ShapeDtypeStruct((2, NROW, DH), jnp.float32)],
    )(x, w_split, deg2)


def _tc_layer2(s1, g1, di, b1_split, w2_split):
    def body(s_ref, g_ref, di_ref, b_ref, w_ref, o_ref):
        d = di_ref[...]
        t0 = jax.nn.relu((s_ref[0] + g_ref[0]) * d + b_ref[0])
        t1 = jax.nn.relu((s_ref[1] + g_ref[1]) * d + b_ref[1])
        t = jnp.concatenate([t0, t1], axis=1)
        o_ref[0] = jnp.dot(t, w_ref[0],
                           preferred_element_type=jnp.float32) * d

    return pl.pallas_call(
        body,
        grid=(NROW // BN, 2),
        in_specs=[pl.BlockSpec((2, BN, DH), lambda i, c: (0, i, 0)),
                  pl.BlockSpec((2, BN, DH), lambda i, c: (0, i, 0)),
                  pl.BlockSpec((BN, 1), lambda i, c: (i, 0)),
                  pl.BlockSpec((2, 1, DH), lambda i, c: (0, 0, 0)),
                  pl.BlockSpec((1, D, DH), lambda i, c: (c, 0, 0))],
        out_specs=pl.BlockSpec((1, BN, DH), lambda i, c: (c, i, 0)),
        out_shape=jax.ShapeDtypeStruct((2, NROW, DH), jnp.float32),
    )(s1, g1, di, b1_split, w2_split)


def _tc_final(s2, g2, di, b2_split):
    def body(s_ref, g_ref, di_ref, b_ref, o_ref):
        d = di_ref[...]
        o0 = (s_ref[0] + g_ref[0]) * d + b_ref[0]
        o1 = (s_ref[1] + g_ref[1]) * d + b_ref[1]
        o_ref[...] = jnp.concatenate([o0, o1], axis=1)

    return pl.pallas_call(
        body,
        grid=(NROW // BN,),
        in_specs=[pl.BlockSpec((2, BN, DH), lambda i: (0, i, 0)),
                  pl.BlockSpec((2, BN, DH), lambda i: (0, i, 0)),
                  pl.BlockSpec((BN, 1), lambda i: (i, 0)),
                  pl.BlockSpec((2, 1, DH), lambda i: (0, 0, 0))],
        out_specs=pl.BlockSpec((BN, D), lambda i: (i, 0)),
        out_shape=jax.ShapeDtypeStruct((NROW, D), jnp.float32),
    )(s2, g2, di, b2_split)


def kernel(x, adj_t, W1, b1, W2, b2):
    src = adj_t[0].astype(jnp.int32)
    dst = adj_t[1].astype(jnp.int32)
    pad = EPAD - E
    src_p = jnp.concatenate([src, jnp.zeros((pad,), jnp.int32)])
    dst_p = jnp.concatenate([dst, jnp.full((pad,), TRASH, jnp.int32)])
    dst3 = dst_p.reshape(NSUB, NCHUNK, CHUNK)
    sd = jnp.concatenate([src_p.reshape(NSUB, NG, GB, CHUNK),
                          dst_p.reshape(NSUB, NG, GB, CHUNK)],
                         axis=2).reshape(NSUB * NG, 2 * GB, CHUNK)

    x_p = jnp.pad(x, ((0, NROW - N), (0, 0)))
    w1s = W1.reshape(D, 2, DH).transpose(1, 0, 2)
    w2s = W2.reshape(D, 2, DH).transpose(1, 0, 2)
    b1s = b1.reshape(2, 1, DH)
    b2s = b2.reshape(2, 1, DH)
    zeros_big = jnp.zeros((NROW, DH), jnp.float32)

    deg_parts = _sc_degree(dst3)
    deg2 = deg_parts.reshape(2, NROW, 1)
    di, g1 = _tc_mm_scale(x_p, w1s, deg2)
    s1 = _sc_scatter(g1, sd, zeros_big)
    g2 = _tc_layer2(s1, g1, di, b1s, w2s)
    s2 = _sc_scatter(g2, sd, zeros_big)
    out = _tc_final(s2, g2, di, b2s)
    return out[:N]


# speedup vs baseline: 1.0394x; 1.0083x over previous
import dataclasses
import functools

import jax
import jax.numpy as jnp
from jax import lax
from jax.experimental import pallas as pl
from jax.experimental.pallas import tpu as pltpu
from jax.experimental.pallas import tpu_sc as plsc

N = 10000
D = 256
DH = 128
NROW = 10240
E = 160000
CHUNK = 128
NCHUNK = 80
GB = 16
NG = NCHUNK // GB
EPAD = 16 * NCHUNK * CHUNK
TRASH = 10200
BN = 1024
NSUB = 16
RPS = NROW // NSUB

_MESH = dict(core_axis_name="c", subcore_axis_name="s")


def _sc_compiler_params():
    cp = pltpu.CompilerParams()
    if "needs_layout_passes" in pltpu.CompilerParams.__dataclass_fields__:
        cp = dataclasses.replace(cp, needs_layout_passes=False)
    return cp


def _sc_degree(dst3):
    mesh = plsc.VectorSubcoreMesh(**_MESH)

    @functools.partial(
        pl.kernel,
        mesh=mesh,
        out_type=jax.ShapeDtypeStruct((2 * NSUB, RPS), jnp.float32),
        compiler_params=_sc_compiler_params(),
        scratch_types=[
            pltpu.VMEM((NCHUNK, CHUNK), jnp.int32),
            pltpu.VMEM((NROW,), jnp.float32),
            pltpu.VMEM((NSUB, RPS), jnp.float32),
            pltpu.VMEM((RPS,), jnp.float32),
            pltpu.VMEM_SHARED((NSUB, NROW), jnp.float32),
        ],
    )
    def k(dst_hbm, out_hbm, dst_v, hist_v, red_v, comp_v, acc):
        cid = lax.axis_index("c")
        sid = lax.axis_index("s")
        one16 = jnp.ones((16,), jnp.float32)
        zero16 = jnp.zeros((16,), jnp.float32)

        @pl.loop(0, NROW // 16)
        def _(i):
            hist_v[pl.ds(i * 16, 16)] = zero16

        pltpu.sync_copy(dst_hbm.at[sid], dst_v)

        half = NCHUNK // 2

        @pl.loop(cid * half, (cid + 1) * half)
        def _(j):
            for kk in range(CHUNK // 16):
                idx = dst_v[j, pl.ds(kk * 16, 16)]
                plsc.addupdate_scatter(hist_v, [idx], one16)

        pltpu.sync_copy(hist_v, acc.at[sid])
        plsc.subcore_barrier()

        for k16 in range(NSUB):
            pltpu.sync_copy(acc.at[k16].at[pl.ds(sid * RPS, RPS)],
                            red_v.at[k16])
        for cc in range(RPS // 16):
            tot = red_v[0, pl.ds(cc * 16, 16)]
            for k16 in range(1, NSUB):
                tot = tot + red_v[k16, pl.ds(cc * 16, 16)]
            comp_v[pl.ds(cc * 16, 16)] = tot

        pltpu.sync_copy(comp_v, out_hbm.at[cid * NSUB + sid])

    return k(dst3)


def _sc_scatter(g, sd, zeros):
    mesh = plsc.VectorSubcoreMesh(**_MESH)

    @functools.partial(
        pl.kernel,
        mesh=mesh,
        out_type=jax.ShapeDtypeStruct((2, NROW, DH), jnp.float32),
        scratch_types=[
            pltpu.VMEM((2 * GB, CHUNK), jnp.int32),
            pltpu.VMEM((2 * GB, CHUNK), jnp.int32),
            pltpu.VMEM((CHUNK, DH), jnp.float32),
            pltpu.VMEM((CHUNK, DH), jnp.float32),
            pltpu.VMEM_SHARED((NROW, DH), jnp.float32),
            pltpu.SemaphoreType.DMA,
            pltpu.SemaphoreType.DMA,
            pltpu.SemaphoreType.DMA,
            pltpu.SemaphoreType.DMA,
            pltpu.SemaphoreType.DMA,
            pltpu.SemaphoreType.DMA,
        ],
    )
    def k(g_hbm, sd_hbm, z_hbm, out_hbm,
          sd_a, sd_b, rows_a, rows_b, acc, sia, sib, sga, sgb, ssa, ssb):
        cid = lax.axis_index("c")
        sid = lax.axis_index("s")
        pltpu.sync_copy(z_hbm.at[pl.ds(sid * RPS, RPS)],
                        acc.at[pl.ds(sid * RPS, RPS)])
        plsc.subcore_barrier()

        gtab = g_hbm.at[cid]

        def idx_cp(grp, buf, sem):
            return pltpu.make_async_copy(sd_hbm.at[sid * NG + grp], buf, sem)

        def gcp(sdbuf, jj, rbuf, sem):
            return pltpu.make_async_copy(gtab.at[sdbuf.at[jj]], rbuf, sem)

        def scp(sdbuf, jj, rbuf, sem):
            return pltpu.make_async_copy(rbuf, acc.at[sdbuf.at[GB + jj]], sem)

        idx_cp(0, sd_a, sia).start()

        def group(grp, sdc, sic, sdn, sin):
            idx_cp(grp, sdc, sic).wait()

            @pl.when(grp + 1 < NG)
            def _():
                idx_cp(grp + 1, sdn, sin).start()

            gcp(sdc, 0, rows_a, sga).start()
            for jj in range(GB):
                even = jj % 2 == 0
                cur, gcur, scur = ((rows_a, sga, ssa) if even
                                   else (rows_b, sgb, ssb))
                oth, goth, soth = ((rows_b, sgb, ssb) if even
                                   else (rows_a, sga, ssa))
                gcp(sdc, jj, cur, gcur).wait()
                if jj + 1 < GB:
                    if jj >= 1:
                        scp(sdc, jj - 1, oth, soth).wait()
                    gcp(sdc, jj + 1, oth, goth).start()
                scp(sdc, jj, cur, scur).start(add=True)
            scp(sdc, GB - 2, rows_a, ssa).wait()
            scp(sdc, GB - 1, rows_b, ssb).wait()

        @pl.loop(0, NG - 1, step=2)
        def _(grp):
            group(grp, sd_a, sia, sd_b, sib)
            group(grp + 1, sd_b, sib, sd_a, sia)

        if NG % 2 == 1:
            group(NG - 1, sd_a, sia, sd_b, sib)

        plsc.subcore_barrier()
        pltpu.sync_copy(acc.at[pl.ds(sid * RPS, RPS)],
                        out_hbm.at[cid].at[pl.ds(sid * RPS, RPS)])

    return k(g, sd, zeros)


def _tc_mm_scale(x, w_split, deg2):
    def body(x_ref, w_ref, dg_ref, di_ref, g_ref):
        d = dg_ref[0] + dg_ref[1] + 1.0
        di = lax.rsqrt(d)
        di_ref[...] = di
        xb = x_ref[...]
        g_ref[0] = jnp.dot(xb, w_ref[0],
                           preferred_element_type=jnp.float32) * di
        g_ref[1] = jnp.dot(xb, w_ref[1],
                           preferred_element_type=jnp.float32) * di

    return pl.pallas_call(
        body,
        grid=(NROW // BN,),
        in_specs=[pl.BlockSpec((BN, D), lambda i: (i, 0)),
                  pl.BlockSpec((2, D, DH), lambda i: (0, 0, 0)),
                  pl.BlockSpec((2, BN, 1), lambda i: (0, i, 0))],
        out_specs=[pl.BlockSpec((BN, 1), lambda i: (i, 0)),
                   pl.BlockSpec((2, BN, DH), lambda i: (0, i, 0))],
        out_shape=[jax.ShapeDtypeStruct((NROW, 1), jnp.float32),
                   jax.ShapeDtypeStruct((2, NROW, DH), jnp.float32)],
    )(x, w_split, deg2)


def _tc_layer2(s1, g1, di, b1_split, w2_split):
    def body(s_ref, g_ref, di_ref, b_ref, w_ref, o_ref):
        d = di_ref[...]
        t0 = jax.nn.relu((s_ref[0] + g_ref[0]) * d + b_ref[0])
        t1 = jax.nn.relu((s_ref[1] + g_ref[1]) * d + b_ref[1])
        t = jnp.concatenate([t0, t1], axis=1)
        o_ref[0] = jnp.dot(t, w_ref[0],
                           preferred_element_type=jnp.float32) * d

    return pl.pallas_call(
        body,
        grid=(NROW // BN, 2),
        in_specs=[pl.BlockSpec((2, BN, DH), lambda i, c: (0, i, 0)),
                  pl.BlockSpec((2, BN, DH), lambda i, c: (0, i, 0)),
                  pl.BlockSpec((BN, 1), lambda i, c: (i, 0)),
                  pl.BlockSpec((2, 1, DH), lambda i, c: (0, 0, 0)),
                  pl.BlockSpec((1, D, DH), lambda i, c: (c, 0, 0))],
        out_specs=pl.BlockSpec((1, BN, DH), lambda i, c: (c, i, 0)),
        out_shape=jax.ShapeDtypeStruct((2, NROW, DH), jnp.float32),
    )(s1, g1, di, b1_split, w2_split)


def _tc_final(s2, g2, di, b2_split):
    def body(s_ref, g_ref, di_ref, b_ref, o_ref):
        d = di_ref[...]
        o0 = (s_ref[0] + g_ref[0]) * d + b_ref[0]
        o1 = (s_ref[1] + g_ref[1]) * d + b_ref[1]
        o_ref[...] = jnp.concatenate([o0, o1], axis=1)

    return pl.pallas_call(
        body,
        grid=(NROW // BN,),
        in_specs=[pl.BlockSpec((2, BN, DH), lambda i: (0, i, 0)),
                  pl.BlockSpec((2, BN, DH), lambda i: (0, i, 0)),
                  pl.BlockSpec((BN, 1), lambda i: (i, 0)),
                  pl.BlockSpec((2, 1, DH), lambda i: (0, 0, 0))],
        out_specs=pl.BlockSpec((BN, D), lambda i: (i, 0)),
        out_shape=jax.ShapeDtypeStruct((NROW, D), jnp.float32),
    )(s2, g2, di, b2_split)


def kernel(x, adj_t, W1, b1, W2, b2):
    src = adj_t[0].astype(jnp.int32)
    dst = adj_t[1].astype(jnp.int32)
    pad = EPAD - E
    src_p = jnp.concatenate([src, jnp.zeros((pad,), jnp.int32)])
    dst_p = jnp.concatenate([dst, jnp.full((pad,), TRASH, jnp.int32)])
    dst3 = dst_p.reshape(NSUB, NCHUNK, CHUNK)
    sd = jnp.concatenate([src_p.reshape(NSUB, NG, GB, CHUNK),
                          dst_p.reshape(NSUB, NG, GB, CHUNK)],
                         axis=2).reshape(NSUB * NG, 2 * GB, CHUNK)

    x_p = jnp.pad(x, ((0, NROW - N), (0, 0)))
    w1s = W1.reshape(D, 2, DH).transpose(1, 0, 2)
    w2s = W2.reshape(D, 2, DH).transpose(1, 0, 2)
    b1s = b1.reshape(2, 1, DH)
    b2s = b2.reshape(2, 1, DH)
    zeros_big = jnp.zeros((NROW, DH), jnp.float32)

    deg_parts = _sc_degree(dst3)
    deg2 = deg_parts.reshape(2, NROW, 1)
    di, g1 = _tc_mm_scale(x_p, w1s, deg2)
    s1 = _sc_scatter(g1, sd, zeros_big)
    g2 = _tc_layer2(s1, g1, di, b1s, w2s)
    s2 = _sc_scatter(g2, sd, zeros_big)
    out = _tc_final(s2, g2, di, b2s)
    return out[:N]
